# baseline (speedup 1.0000x reference)
import functools

import jax
import jax.numpy as jnp
from jax import lax
from jax.experimental import pallas as pl
from jax.experimental.pallas import tpu as pltpu
from jax.experimental.pallas import tpu_sc as plsc

E = 8
KTOP = 2
H = 1024
I = 4096
T = 4096

BT = 1024
SUB = 128
NSUB = BT // SUB
BI = 1024
NI = I // BI
NB = (T * KTOP + E * (BT - 1) + (BT - 1)) // BT
P = NB * BT
TB_R = 1024

NC = 2
NS = 16
NW = NC * NS
TPW = T // NW
DCH = 64
CCH = 32


def _router_body(x_ref, gw_ref, idx_ref, w_ref):
    x = x_ref[...]
    gw = gw_ref[...]
    logits = lax.dot_general(x, gw, (((1,), (1,)), ((), ())),
                             preferred_element_type=jnp.float32)
    m0 = jnp.max(logits, axis=1, keepdims=True)
    ids = lax.broadcasted_iota(jnp.int32, logits.shape, 1)
    is0 = logits == m0
    i0 = jnp.min(jnp.where(is0, ids, E), axis=1, keepdims=True)
    masked = jnp.where(ids == i0, -jnp.inf, logits)
    m1 = jnp.max(masked, axis=1, keepdims=True)
    is1 = masked == m1
    i1 = jnp.min(jnp.where(is1, ids, E), axis=1, keepdims=True)
    e1 = jnp.exp(m1 - m0)
    w0 = 1.0 / (1.0 + e1)
    w1 = e1 / (1.0 + e1)
    idx_ref[...] = jnp.concatenate([i0, i1], axis=1)
    w_ref[...] = jnp.concatenate([w0, w1], axis=1)


def _router(hidden, gate_w):
    return pl.pallas_call(
        _router_body,
        grid=(T // TB_R,),
        in_specs=[
            pl.BlockSpec((TB_R, H), lambda b: (b, 0)),
            pl.BlockSpec((E, H), lambda b: (0, 0)),
        ],
        out_specs=[
            pl.BlockSpec((TB_R, KTOP), lambda b: (b, 0)),
            pl.BlockSpec((TB_R, KTOP), lambda b: (b, 0)),
        ],
        out_shape=[
            jax.ShapeDtypeStruct((T, KTOP), jnp.int32),
            jax.ShapeDtypeStruct((T, KTOP), jnp.float32),
        ],
    )(hidden, gate_w)


def _dispatch_body(hidden_hbm, d0_hbm, d1_hbm, x_hbm, idx_v, rows_v, sem):
    wid = lax.axis_index("s") * NC + lax.axis_index("c")
    base = wid * TPW
    for c in range(TPW // DCH):
        tb = base + c * DCH
        pltpu.sync_copy(hidden_hbm.at[pl.ds(tb, DCH)], rows_v)
        pltpu.sync_copy(d0_hbm.at[pl.ds(tb, DCH)], idx_v)
        pltpu.async_copy(rows_v, x_hbm.at[idx_v], sem).wait()
        pltpu.sync_copy(d1_hbm.at[pl.ds(tb, DCH)], idx_v)
        pltpu.async_copy(rows_v, x_hbm.at[idx_v], sem).wait()


_dispatch = functools.partial(
    pl.kernel,
    _dispatch_body,
    out_type=jax.ShapeDtypeStruct((P, H), jnp.float32),
    mesh=plsc.VectorSubcoreMesh(core_axis_name="c", subcore_axis_name="s",
                                num_cores=NC),
    scratch_types=[
        pltpu.VMEM((DCH,), jnp.int32),
        pltpu.VMEM((DCH, H), jnp.float32),
        pltpu.SemaphoreType.DMA,
    ],
)()


def _combine_body(y_hbm, p0_hbm, p1_hbm, w0_hbm, w1_hbm, o_hbm,
                  i0_v, i1_v, a_v, b_v, wa_v, wb_v, sem0, sem1):
    wid = lax.axis_index("s") * NC + lax.axis_index("c")
    base = wid * TPW
    for c in range(TPW // CCH):
        tb = base + c * CCH
        pltpu.sync_copy(p0_hbm.at[pl.ds(tb, CCH)], i0_v)
        pltpu.sync_copy(p1_hbm.at[pl.ds(tb, CCH)], i1_v)
        cpa = pltpu.async_copy(y_hbm.at[i0_v], a_v, sem0)
        cpb = pltpu.async_copy(y_hbm.at[i1_v], b_v, sem1)
        pltpu.sync_copy(w0_hbm.at[pl.ds(tb, CCH)], wa_v)
        pltpu.sync_copy(w1_hbm.at[pl.ds(tb, CCH)], wb_v)
        cpa.wait()
        cpb.wait()

        def _row(r, carry):
            wa = wa_v[r, :]
            wb = wb_v[r, :]

            @plsc.parallel_loop(0, H // 16, 1, unroll=8)
            def _col(k):
                sl = pl.ds(k * 16, 16)
                a_v[r, sl] = a_v[r, sl] * wa + b_v[r, sl] * wb

            return carry

        lax.fori_loop(0, CCH, _row, 0)
        pltpu.sync_copy(a_v, o_hbm.at[pl.ds(tb, CCH)])


_combine = functools.partial(
    pl.kernel,
    _combine_body,
    out_type=jax.ShapeDtypeStruct((T, H), jnp.float32),
    mesh=plsc.VectorSubcoreMesh(core_axis_name="c", subcore_axis_name="s",
                                num_cores=NC),
    scratch_types=[
        pltpu.VMEM((CCH,), jnp.int32),
        pltpu.VMEM((CCH,), jnp.int32),
        pltpu.VMEM((CCH, H), jnp.float32),
        pltpu.VMEM((CCH, H), jnp.float32),
        pltpu.VMEM((CCH, 16), jnp.float32),
        pltpu.VMEM((CCH, 16), jnp.float32),
        pltpu.SemaphoreType.DMA,
        pltpu.SemaphoreType.DMA,
    ],
)()


def _mm_body(be_ref, bx_ref, uf_ref, su_ref,
             x_ref, wg_ref, wu_ref, wd_ref, o_ref):
    del be_ref, bx_ref, uf_ref
    b = pl.program_id(0)
    i = pl.program_id(1)
    wg = wg_ref[0]
    wu = wu_ref[0]
    wd = wd_ref[0]
    for s in range(NSUB):
        @pl.when(su_ref[b, s] == 1)
        def _compute(s=s):
            xs = x_ref[pl.ds(s * SUB, SUB), :]
            g = lax.dot_general(xs, wg, (((1,), (1,)), ((), ())),
                                preferred_element_type=jnp.float32)
            u = lax.dot_general(xs, wu, (((1,), (1,)), ((), ())),
                                preferred_element_type=jnp.float32)
            h = (g * lax.logistic(g)) * u
            part = lax.dot_general(h, wd, (((1,), (1,)), ((), ())),
                                   preferred_element_type=jnp.float32)

            @pl.when(i == 0)
            def _init():
                o_ref[pl.ds(s * SUB, SUB), :] = part

            @pl.when(i > 0)
            def _acc():
                o_ref[pl.ds(s * SUB, SUB), :] += part


def _grouped_mm(be, bx, uf, su, x_sorted, gate_ws, up_ws, down_ws):
    grid_spec = pltpu.PrefetchScalarGridSpec(
        num_scalar_prefetch=4,
        grid=(NB, NI),
        in_specs=[
            pl.BlockSpec((BT, H), lambda b, i, be, bx, uf, su: (bx[b], 0)),
            pl.BlockSpec((1, BI, H),
                         lambda b, i, be, bx, uf, su:
                         (be[b], jnp.where(uf[b] == 1, i, NI - 1), 0)),
            pl.BlockSpec((1, BI, H),
                         lambda b, i, be, bx, uf, su:
                         (be[b], jnp.where(uf[b] == 1, i, NI - 1), 0)),
            pl.BlockSpec((1, H, BI),
                         lambda b, i, be, bx, uf, su:
                         (be[b], 0, jnp.where(uf[b] == 1, i, NI - 1))),
        ],
        out_specs=pl.BlockSpec((BT, H), lambda b, i, be, bx, uf, su: (bx[b], 0)),
    )
    return pl.pallas_call(
        _mm_body,
        grid_spec=grid_spec,
        out_shape=jax.ShapeDtypeStruct((P, H), jnp.float32),
        compiler_params=pltpu.CompilerParams(
            dimension_semantics=("arbitrary", "arbitrary"),
        ),
    )(be, bx, uf, su, x_sorted, gate_ws, up_ws, down_ws)


def kernel(hidden_states, gate_w, gate_ws, up_ws, down_ws):
    top_idx, rw = _router(hidden_states, gate_w)

    e_flat = top_idx.reshape(-1)
    onehot = (e_flat[:, None] == jnp.arange(E)[None, :]).astype(jnp.int32)
    counts = jnp.sum(onehot, axis=0)
    rank = jnp.sum(jnp.where(onehot != 0,
                             jnp.cumsum(onehot, axis=0) - 1, 0), axis=1)
    padded = ((counts + BT - 1) // BT) * BT
    seg_end = jnp.cumsum(padded)
    seg_start = seg_end - padded
    dest = (seg_start[e_flat] + rank).astype(jnp.int32)

    blocks = jnp.arange(NB, dtype=jnp.int32)
    be = jnp.minimum(
        jnp.searchsorted(seg_end, blocks * BT, side="right"), E - 1
    ).astype(jnp.int32)
    used_rows = jnp.clip(counts[be] - (blocks * BT - seg_start[be]), 0, BT)
    uf = (used_rows > 0).astype(jnp.int32)
    n_used = jnp.maximum(jnp.sum(uf), 1)
    bx = jnp.minimum(blocks, n_used - 1).astype(jnp.int32)
    su = (used_rows[:, None]
          > jnp.arange(NSUB, dtype=jnp.int32)[None, :] * SUB).astype(jnp.int32)

    pos = dest.reshape(T, KTOP)
    p0 = pos[:, 0]
    p1 = pos[:, 1]
    x_sorted = _dispatch(hidden_states, p0, p1)

    y = _grouped_mm(be, bx, uf, su, x_sorted, gate_ws, up_ws, down_ws)

    w_wide = jnp.broadcast_to(rw[:, :, None], (T, KTOP, 16))
    return _combine(y, p0, p1, w_wide[:, 0, :], w_wide[:, 1, :])


# speedup vs baseline: 1.7746x; 1.7746x over previous
import functools

import jax
import jax.numpy as jnp
from jax import lax
from jax.experimental import pallas as pl
from jax.experimental.pallas import tpu as pltpu
from jax.experimental.pallas import tpu_sc as plsc

E = 8
KTOP = 2
H = 1024
I = 4096
T = 4096

BT = 1024
SUB = 512
NSUB = BT // SUB
BI = 1024
NI = I // BI
NB = (T * KTOP + E * (BT - 1) + (BT - 1)) // BT
P = NB * BT
TB_R = 1024

NC = 2
NS = 16
NW = NC * NS
TPW = T // NW
DCH = 64
CCH = 32


def _router_body(x_ref, gw_ref, idx_ref, w_ref):
    x = x_ref[...]
    gw = gw_ref[...]
    logits = lax.dot_general(x, gw, (((1,), (1,)), ((), ())),
                             preferred_element_type=jnp.float32)
    m0 = jnp.max(logits, axis=1, keepdims=True)
    ids = lax.broadcasted_iota(jnp.int32, logits.shape, 1)
    is0 = logits == m0
    i0 = jnp.min(jnp.where(is0, ids, E), axis=1, keepdims=True)
    masked = jnp.where(ids == i0, -jnp.inf, logits)
    m1 = jnp.max(masked, axis=1, keepdims=True)
    is1 = masked == m1
    i1 = jnp.min(jnp.where(is1, ids, E), axis=1, keepdims=True)
    e1 = jnp.exp(m1 - m0)
    w0 = 1.0 / (1.0 + e1)
    w1 = e1 / (1.0 + e1)
    idx_ref[...] = jnp.concatenate([i0, i1], axis=1)
    w_ref[...] = jnp.concatenate([w0, w1], axis=1)


def _router(hidden, gate_w):
    return pl.pallas_call(
        _router_body,
        grid=(T // TB_R,),
        in_specs=[
            pl.BlockSpec((TB_R, H), lambda b: (b, 0)),
            pl.BlockSpec((E, H), lambda b: (0, 0)),
        ],
        out_specs=[
            pl.BlockSpec((TB_R, KTOP), lambda b: (b, 0)),
            pl.BlockSpec((TB_R, KTOP), lambda b: (b, 0)),
        ],
        out_shape=[
            jax.ShapeDtypeStruct((T, KTOP), jnp.int32),
            jax.ShapeDtypeStruct((T, KTOP), jnp.float32),
        ],
    )(hidden, gate_w)


def _dispatch_body(hidden_hbm, d0_hbm, d1_hbm, x_hbm, idx_v, rows_v, sem):
    wid = lax.axis_index("s") * NC + lax.axis_index("c")
    base = wid * TPW
    for c in range(TPW // DCH):
        tb = base + c * DCH
        pltpu.sync_copy(hidden_hbm.at[pl.ds(tb, DCH)], rows_v)
        pltpu.sync_copy(d0_hbm.at[pl.ds(tb, DCH)], idx_v)
        pltpu.async_copy(rows_v, x_hbm.at[idx_v], sem).wait()
        pltpu.sync_copy(d1_hbm.at[pl.ds(tb, DCH)], idx_v)
        pltpu.async_copy(rows_v, x_hbm.at[idx_v], sem).wait()


_dispatch = functools.partial(
    pl.kernel,
    _dispatch_body,
    out_type=jax.ShapeDtypeStruct((P, H), jnp.float32),
    mesh=plsc.VectorSubcoreMesh(core_axis_name="c", subcore_axis_name="s",
                                num_cores=NC),
    scratch_types=[
        pltpu.VMEM((DCH,), jnp.int32),
        pltpu.VMEM((DCH, H), jnp.float32),
        pltpu.SemaphoreType.DMA,
    ],
)()


def _combine_body(y_hbm, p0_hbm, p1_hbm, w0_hbm, w1_hbm, o_hbm,
                  i0_v, i1_v, a_v, b_v, wa_v, wb_v, sem0, sem1):
    wid = lax.axis_index("s") * NC + lax.axis_index("c")
    base = wid * TPW
    for c in range(TPW // CCH):
        tb = base + c * CCH
        pltpu.sync_copy(p0_hbm.at[pl.ds(tb, CCH)], i0_v)
        pltpu.sync_copy(p1_hbm.at[pl.ds(tb, CCH)], i1_v)
        cpa = pltpu.async_copy(y_hbm.at[i0_v], a_v, sem0)
        cpb = pltpu.async_copy(y_hbm.at[i1_v], b_v, sem1)
        pltpu.sync_copy(w0_hbm.at[pl.ds(tb, CCH)], wa_v)
        pltpu.sync_copy(w1_hbm.at[pl.ds(tb, CCH)], wb_v)
        cpa.wait()
        cpb.wait()

        def _row(r, carry):
            wa = wa_v[r, :]
            wb = wb_v[r, :]

            @plsc.parallel_loop(0, H // 16, 1, unroll=8)
            def _col(k):
                sl = pl.ds(k * 16, 16)
                a_v[r, sl] = a_v[r, sl] * wa + b_v[r, sl] * wb

            return carry

        lax.fori_loop(0, CCH, _row, 0)
        pltpu.sync_copy(a_v, o_hbm.at[pl.ds(tb, CCH)])


_combine = functools.partial(
    pl.kernel,
    _combine_body,
    out_type=jax.ShapeDtypeStruct((T, H), jnp.float32),
    mesh=plsc.VectorSubcoreMesh(core_axis_name="c", subcore_axis_name="s",
                                num_cores=NC),
    scratch_types=[
        pltpu.VMEM((CCH,), jnp.int32),
        pltpu.VMEM((CCH,), jnp.int32),
        pltpu.VMEM((CCH, H), jnp.float32),
        pltpu.VMEM((CCH, H), jnp.float32),
        pltpu.VMEM((CCH, 16), jnp.float32),
        pltpu.VMEM((CCH, 16), jnp.float32),
        pltpu.SemaphoreType.DMA,
        pltpu.SemaphoreType.DMA,
    ],
)()


def _mm_body(be_ref, bx_ref, uf_ref, su_ref,
             x_ref, wg_ref, wu_ref, wd_ref, o_ref):
    del be_ref, bx_ref, uf_ref
    b = pl.program_id(0)
    i = pl.program_id(1)
    wg = wg_ref[0]
    wu = wu_ref[0]
    wd = wd_ref[0]
    for s in range(NSUB):
        @pl.when(su_ref[b, s] == 1)
        def _compute(s=s):
            xs = x_ref[pl.ds(s * SUB, SUB), :]
            g = lax.dot_general(xs, wg, (((1,), (1,)), ((), ())),
                                preferred_element_type=jnp.float32)
            u = lax.dot_general(xs, wu, (((1,), (1,)), ((), ())),
                                preferred_element_type=jnp.float32)
            h = (g * lax.logistic(g)) * u
            part = lax.dot_general(h, wd, (((1,), (1,)), ((), ())),
                                   preferred_element_type=jnp.float32)

            @pl.when(i == 0)
            def _init():
                o_ref[pl.ds(s * SUB, SUB), :] = part

            @pl.when(i > 0)
            def _acc():
                o_ref[pl.ds(s * SUB, SUB), :] += part


def _grouped_mm(be, bx, uf, su, x_sorted, gate_ws, up_ws, down_ws):
    grid_spec = pltpu.PrefetchScalarGridSpec(
        num_scalar_prefetch=4,
        grid=(NB, NI),
        in_specs=[
            pl.BlockSpec((BT, H), lambda b, i, be, bx, uf, su: (bx[b], 0)),
            pl.BlockSpec((1, BI, H),
                         lambda b, i, be, bx, uf, su:
                         (be[b], jnp.where(uf[b] == 1, i, NI - 1), 0)),
            pl.BlockSpec((1, BI, H),
                         lambda b, i, be, bx, uf, su:
                         (be[b], jnp.where(uf[b] == 1, i, NI - 1), 0)),
            pl.BlockSpec((1, H, BI),
                         lambda b, i, be, bx, uf, su:
                         (be[b], 0, jnp.where(uf[b] == 1, i, NI - 1))),
        ],
        out_specs=pl.BlockSpec((BT, H), lambda b, i, be, bx, uf, su: (bx[b], 0)),
    )
    return pl.pallas_call(
        _mm_body,
        grid_spec=grid_spec,
        out_shape=jax.ShapeDtypeStruct((P, H), jnp.float32),
        compiler_params=pltpu.CompilerParams(
            dimension_semantics=("arbitrary", "arbitrary"),
        ),
    )(be, bx, uf, su, x_sorted, gate_ws, up_ws, down_ws)


def kernel(hidden_states, gate_w, gate_ws, up_ws, down_ws):
    top_idx, rw = _router(hidden_states, gate_w)

    e_flat = top_idx.reshape(-1)
    onehot = (e_flat[:, None] == jnp.arange(E)[None, :]).astype(jnp.int32)
    counts = jnp.sum(onehot, axis=0)
    rank = jnp.sum(jnp.where(onehot != 0,
                             jnp.cumsum(onehot, axis=0) - 1, 0), axis=1)
    padded = ((counts + BT - 1) // BT) * BT
    seg_end = jnp.cumsum(padded)
    seg_start = seg_end - padded
    dest = (seg_start[e_flat] + rank).astype(jnp.int32)

    blocks = jnp.arange(NB, dtype=jnp.int32)
    be = jnp.minimum(
        jnp.searchsorted(seg_end, blocks * BT, side="right"), E - 1
    ).astype(jnp.int32)
    used_rows = jnp.clip(counts[be] - (blocks * BT - seg_start[be]), 0, BT)
    uf = (used_rows > 0).astype(jnp.int32)
    n_used = jnp.maximum(jnp.sum(uf), 1)
    bx = jnp.minimum(blocks, n_used - 1).astype(jnp.int32)
    su = (used_rows[:, None]
          > jnp.arange(NSUB, dtype=jnp.int32)[None, :] * SUB).astype(jnp.int32)

    pos = dest.reshape(T, KTOP)
    p0 = pos[:, 0]
    p1 = pos[:, 1]
    x_sorted = _dispatch(hidden_states, p0, p1)

    y = _grouped_mm(be, bx, uf, su, x_sorted, gate_ws, up_ws, down_ws)

    w_wide = jnp.broadcast_to(rw[:, :, None], (T, KTOP, 16))
    return _combine(y, p0, p1, w_wide[:, 0, :], w_wide[:, 1, :])
